# baseline (speedup 1.0000x reference)
import functools

import jax
import jax.numpy as jnp
from jax import lax
from jax.experimental import pallas as pl
from jax.experimental.pallas import tpu as pltpu
from jax.experimental.pallas import tpu_sc as plsc

_DIM = 128
_K = 65536
_BATCH = 4096
_NW = 32
_ROWS = _K * _DIM // 128
_RPW = _ROWS // _NW
_CHUNK = 256
_NCHUNK = _RPW // _CHUNK

_MESH = plsc.VectorSubcoreMesh(core_axis_name="c", subcore_axis_name="s")


@functools.partial(
    pl.kernel,
    out_type=jax.ShapeDtypeStruct((_ROWS, 128), jnp.float32),
    mesh=_MESH,
    scratch_types=[
        pltpu.VMEM((2, _CHUNK, 128), jnp.float32),
        pltpu.SemaphoreType.DMA,
        pltpu.SemaphoreType.DMA,
        pltpu.SemaphoreType.DMA,
        pltpu.SemaphoreType.DMA,
    ],
)
def _sc_copy(q_hbm, out_hbm, bufs, si0, si1, so0, so1):
    w = lax.axis_index("s") * 2 + lax.axis_index("c")
    base = w * _RPW
    sin = (si0, si1)
    sout = (so0, so1)
    out_handles = [None, None]
    in_handles = [None, None]
    for c in range(_NCHUNK):
        b = c % 2
        if out_handles[b] is not None:
            out_handles[b].wait()
        in_handles[b] = pltpu.async_copy(
            q_hbm.at[pl.ds(base + c * _CHUNK, _CHUNK), :], bufs.at[b], sin[b]
        )
        in_handles[b].wait()
        out_handles[b] = pltpu.async_copy(
            bufs.at[b], out_hbm.at[pl.ds(base + c * _CHUNK, _CHUNK), :], sout[b]
        )
    out_handles[0].wait()
    out_handles[1].wait()


@functools.partial(
    pl.kernel,
    out_type=(),
    mesh=_MESH,
    scratch_types=[
        pltpu.VMEM((128, 128), jnp.float32),
        pltpu.VMEM((128, 128), jnp.float32),
        pltpu.VMEM((128,), jnp.int32),
        pltpu.VMEM((128,), jnp.int32),
        pltpu.SemaphoreType.DMA,
    ],
    compiler_params=pltpu.CompilerParams(needs_layout_passes=False),
)
def _sc_scatter(keys_hbm, ibase_hbm, out_ref, kt, tt, idxv, ibv, sem):
    w = lax.axis_index("s") * 2 + lax.axis_index("c")
    pltpu.sync_copy(keys_hbm.at[pl.ds(w * 128, 128), :], kt)
    pltpu.sync_copy(ibase_hbm, ibv)
    lane = lax.iota(jnp.int32, 16)

    def tbody(d, _):
        dvec = jnp.full((16,), d, jnp.int32)
        for c in range(8):
            rvec = c * 16 + lane
            vals = plsc.load_gather(kt, [rvec, dvec])
            plsc.store_scatter(tt, [dvec, rvec], vals)
        return _

    lax.fori_loop(0, 128, tbody, 0)
    for c in range(8):
        idxv[pl.ds(c * 16, 16)] = ibv[pl.ds(c * 16, 16)] + w
    pltpu.async_copy(tt, out_ref.at[idxv], sem).wait()


def kernel(keys, queue, queue_ptr):
    ptr = jnp.asarray(queue_ptr, jnp.int32)
    qv = queue.reshape(_ROWS, 128)
    out0 = _sc_copy(qv)
    ibase = lax.iota(jnp.int32, 128) * 512 + ptr // 128
    r = jax.new_ref(out0)
    _sc_scatter(keys, ibase, r)
    new_queue = r[...].reshape(_DIM, _K)
    new_ptr = (ptr + _BATCH) % _K
    return new_queue, jnp.asarray(new_ptr, dtype=jnp.int64)
